# Initial kernel scaffold; baseline (speedup 1.0000x reference)
import jax
import jax.numpy as jnp
from jax.experimental import pallas as pl


def kernel(X_genre, X_history, X_history_ratings, timestamps, target_genre, target_year, target_book_idx, target_author_idx, item_table, author_table, year_table, ts_table, shelf_matrix, W_item, b_item, W_auth, b_auth, W_shelf, b_shelf, W_ig, b_ig, W_yr, b_yr, W_ug, b_ug, W_ts, b_ts):
    raise NotImplementedError("write your pallas kernel here")


# speedup vs baseline: 3.3250x; 3.3250x over previous
import functools

import jax
import jax.numpy as jnp
from jax import lax
from jax.experimental import pallas as pl
from jax.experimental.pallas import tpu as pltpu
from jax.experimental.pallas import tpu_sc as plsc

N_BOOKS = 100000
B = 4096
H = 200
D_ITEM = 64
N_SHELVES = 64
D_AUTH = 16
D_YEAR = 16
D_TS = 32

NC = 2
NS = 16
NW = NC * NS
ROWS_PER_W = B // NW
H_PAD = 208
HA = 112
HB = H_PAD - HA


def _sc_body(item_hbm, shelf_hbm, auth_hbm, year_hbm, ts_hbm,
             xh_hbm, rat_hbm, tb_hbm, ta_hbm, ty_hbm, tt_hbm,
             hist_out, wsum_out, titem_out, tshelf_out, tauth_out,
             tyear_out, tts_out,
             idx128, buf64, buf16a, buf16b, buf32,
             idxA, idxB, ratA, ratB, rowsA, rowsB, histbuf, wsbuf,
             sem, semA, semB):
    wid = lax.axis_index("s") * NC + lax.axis_index("c")
    base = wid * ROWS_PER_W

    pltpu.sync_copy(tb_hbm.at[pl.ds(base, ROWS_PER_W)], idx128)
    pltpu.async_copy(item_hbm.at[idx128], buf64, sem).wait()
    pltpu.sync_copy(buf64, titem_out.at[pl.ds(base, ROWS_PER_W)])
    pltpu.async_copy(shelf_hbm.at[idx128], buf64, sem).wait()
    pltpu.sync_copy(buf64, tshelf_out.at[pl.ds(base, ROWS_PER_W)])

    pltpu.sync_copy(ta_hbm.at[pl.ds(base, ROWS_PER_W)], idx128)
    pltpu.async_copy(auth_hbm.at[idx128], buf16a, sem).wait()
    pltpu.sync_copy(buf16a, tauth_out.at[pl.ds(base, ROWS_PER_W)])

    pltpu.sync_copy(ty_hbm.at[pl.ds(base, ROWS_PER_W)], idx128)
    pltpu.async_copy(year_hbm.at[idx128], buf16b, sem).wait()
    pltpu.sync_copy(buf16b, tyear_out.at[pl.ds(base, ROWS_PER_W)])

    pltpu.sync_copy(tt_hbm.at[pl.ds(base, ROWS_PER_W)], idx128)
    pltpu.async_copy(ts_hbm.at[idx128], buf32, sem).wait()
    pltpu.sync_copy(buf32, tts_out.at[pl.ds(base, ROWS_PER_W)])

    def row_step(g, _):
        b = base + g
        off = b * H_PAD
        pltpu.sync_copy(xh_hbm.at[pl.ds(off, HA)], idxA)
        pltpu.sync_copy(xh_hbm.at[pl.ds(off + HA, HB)], idxB)
        pltpu.sync_copy(rat_hbm.at[pl.ds(off, HA)], ratA)
        pltpu.sync_copy(rat_hbm.at[pl.ds(off + HA, HB)], ratB)
        cpA = pltpu.async_copy(item_hbm.at[idxA], rowsA, semA)
        cpB = pltpu.async_copy(item_hbm.at[idxB], rowsB, semB)
        cpA.wait()
        cpB.wait()

        def make_chunk_step(idx_ref, rat_ref, rows_ref):
            def chunk_step(k, carry):
                a0, a1, a2, a3, wsv = carry
                h0 = k * 16
                iv = idx_ref[pl.ds(h0, 16)]
                rv = rat_ref[pl.ds(h0, 16)]
                wv = jnp.where(iv != N_BOOKS, rv, jnp.float32(0.0))
                wsv = wsv + jnp.abs(wv)
                for j in range(16):
                    w = wv[j]
                    h = h0 + j
                    a0 = a0 + rows_ref[h, pl.ds(0, 16)] * w
                    a1 = a1 + rows_ref[h, pl.ds(16, 16)] * w
                    a2 = a2 + rows_ref[h, pl.ds(32, 16)] * w
                    a3 = a3 + rows_ref[h, pl.ds(48, 16)] * w
                return a0, a1, a2, a3, wsv
            return chunk_step

        z = jnp.zeros((16,), jnp.float32)
        carry = (z, z, z, z, z)
        carry = lax.fori_loop(0, HA // 16, make_chunk_step(idxA, ratA, rowsA),
                              carry)
        a0, a1, a2, a3, wsv = lax.fori_loop(
            0, HB // 16, make_chunk_step(idxB, ratB, rowsB), carry)
        histbuf[g, pl.ds(0, 16)] = a0
        histbuf[g, pl.ds(16, 16)] = a1
        histbuf[g, pl.ds(32, 16)] = a2
        histbuf[g, pl.ds(48, 16)] = a3
        wsbuf[g, pl.ds(0, 16)] = wsv
        return _

    lax.fori_loop(0, ROWS_PER_W, row_step, 0)
    pltpu.sync_copy(histbuf, hist_out.at[pl.ds(base, ROWS_PER_W)])
    pltpu.sync_copy(wsbuf, wsum_out.at[pl.ds(base, ROWS_PER_W)])


@jax.jit
def _sc_gather(item_table, shelf_matrix, author_table, year_table, ts_table,
               x_history, ratings, tb_idx, ta_idx, ty_idx, tt_idx):
    mesh = plsc.VectorSubcoreMesh(core_axis_name="c", subcore_axis_name="s")
    f = pl.kernel(
        _sc_body,
        out_type=[
            jax.ShapeDtypeStruct((B, D_ITEM), jnp.float32),
            jax.ShapeDtypeStruct((B, 16), jnp.float32),
            jax.ShapeDtypeStruct((B, D_ITEM), jnp.float32),
            jax.ShapeDtypeStruct((B, N_SHELVES), jnp.float32),
            jax.ShapeDtypeStruct((B, D_AUTH), jnp.float32),
            jax.ShapeDtypeStruct((B, D_YEAR), jnp.float32),
            jax.ShapeDtypeStruct((B, D_TS), jnp.float32),
        ],
        mesh=mesh,
        compiler_params=pltpu.CompilerParams(use_tc_tiling_on_sc=False),
        scratch_types=[
            pltpu.VMEM((ROWS_PER_W,), jnp.int32),
            pltpu.VMEM((ROWS_PER_W, D_ITEM), jnp.float32),
            pltpu.VMEM((ROWS_PER_W, D_AUTH), jnp.float32),
            pltpu.VMEM((ROWS_PER_W, D_YEAR), jnp.float32),
            pltpu.VMEM((ROWS_PER_W, D_TS), jnp.float32),
            pltpu.VMEM((HA,), jnp.int32),
            pltpu.VMEM((HB,), jnp.int32),
            pltpu.VMEM((HA,), jnp.float32),
            pltpu.VMEM((HB,), jnp.float32),
            pltpu.VMEM((HA, D_ITEM), jnp.float32),
            pltpu.VMEM((HB, D_ITEM), jnp.float32),
            pltpu.VMEM((ROWS_PER_W, D_ITEM), jnp.float32),
            pltpu.VMEM((ROWS_PER_W, 16), jnp.float32),
            pltpu.SemaphoreType.DMA,
            pltpu.SemaphoreType.DMA,
            pltpu.SemaphoreType.DMA,
        ],
    )
    return f(item_table, shelf_matrix, author_table, year_table, ts_table,
             x_history, ratings, tb_idx, ta_idx, ty_idx, tt_idx)


def _tc_body(hist_ref, wsum_ref, xg_ref, tg_ref, tts_ref, titem_ref,
             tshelf_ref, tauth_ref, tyear_ref,
             wug_ref, bug_ref, wts_ref, bts_ref, wig_ref, big_ref,
             wsh_ref, bsh_ref, wit_ref, bit_ref, wau_ref, bau_ref,
             wyr_ref, byr_ref, out_ref):
    wsum = jnp.maximum(jnp.sum(wsum_ref[...], axis=1, keepdims=True), 1e-6)
    hist = hist_ref[...] / wsum
    g = jnp.tanh(jnp.dot(xg_ref[...], wug_ref[...],
                         preferred_element_type=jnp.float32) + bug_ref[...])
    ts_e = jnp.tanh(jnp.dot(tts_ref[...], wts_ref[...],
                            preferred_element_type=jnp.float32) + bts_ref[...])
    ig = jnp.tanh(jnp.dot(tg_ref[...], wig_ref[...],
                          preferred_element_type=jnp.float32) + big_ref[...])
    sh = jnp.tanh(jnp.dot(tshelf_ref[...], wsh_ref[...],
                          preferred_element_type=jnp.float32) + bsh_ref[...])
    it = jnp.tanh(jnp.dot(titem_ref[...], wit_ref[...],
                          preferred_element_type=jnp.float32) + bit_ref[...])
    au = jnp.tanh(jnp.dot(tauth_ref[...], wau_ref[...],
                          preferred_element_type=jnp.float32) + bau_ref[...])
    yr = jnp.tanh(jnp.dot(tyear_ref[...], wyr_ref[...],
                          preferred_element_type=jnp.float32) + byr_ref[...])
    dot = (jnp.sum(hist[:, :32] * ig, axis=1, keepdims=True)
           + jnp.sum(hist[:, 32:] * sh, axis=1, keepdims=True)
           + jnp.sum(g * it, axis=1, keepdims=True)
           + jnp.sum(ts_e[:, :16] * au, axis=1, keepdims=True)
           + jnp.sum(ts_e[:, 16:] * yr, axis=1, keepdims=True))
    out_ref[...] = dot


def kernel(X_genre, X_history, X_history_ratings, timestamps, target_genre,
           target_year, target_book_idx, target_author_idx, item_table,
           author_table, year_table, ts_table, shelf_matrix, W_item, b_item,
           W_auth, b_auth, W_shelf, b_shelf, W_ig, b_ig, W_yr, b_yr, W_ug,
           b_ug, W_ts, b_ts):
    xh = jnp.pad(X_history.astype(jnp.int32), ((0, 0), (0, H_PAD - H)),
                 constant_values=N_BOOKS).reshape(B * H_PAD)
    rat = jnp.pad(X_history_ratings, ((0, 0), (0, H_PAD - H))).reshape(B * H_PAD)
    tb = target_book_idx.astype(jnp.int32)
    ta = target_author_idx.astype(jnp.int32)
    ty = target_year.astype(jnp.int32)
    tt = timestamps.astype(jnp.int32)

    (hist_raw, wsum, titem, tshelf, tauth, tyear, tts) = _sc_gather(
        item_table, shelf_matrix, author_table, year_table, ts_table,
        xh, rat, tb, ta, ty, tt)

    nblk = 8
    bs = B // nblk
    rep = lambda shape: pl.BlockSpec(shape, lambda i: (0,) * len(shape))
    blk = lambda d: pl.BlockSpec((bs, d), lambda i: (i, 0))
    out = pl.pallas_call(
        _tc_body,
        grid=(nblk,),
        in_specs=[
            blk(D_ITEM), blk(16), blk(X_genre.shape[1]), blk(target_genre.shape[1]),
            blk(D_TS), blk(D_ITEM), blk(N_SHELVES), blk(D_AUTH), blk(D_YEAR),
            rep(W_ug.shape), rep((1, b_ug.shape[0])),
            rep(W_ts.shape), rep((1, b_ts.shape[0])),
            rep(W_ig.shape), rep((1, b_ig.shape[0])),
            rep(W_shelf.shape), rep((1, b_shelf.shape[0])),
            rep(W_item.shape), rep((1, b_item.shape[0])),
            rep(W_auth.shape), rep((1, b_auth.shape[0])),
            rep(W_yr.shape), rep((1, b_yr.shape[0])),
        ],
        out_specs=blk(1),
        out_shape=jax.ShapeDtypeStruct((B, 1), jnp.float32),
    )(hist_raw, wsum, X_genre, target_genre, tts, titem,
      tshelf, tauth, tyear,
      W_ug, b_ug.reshape(1, -1), W_ts, b_ts.reshape(1, -1),
      W_ig, b_ig.reshape(1, -1), W_shelf, b_shelf.reshape(1, -1),
      W_item, b_item.reshape(1, -1), W_auth, b_auth.reshape(1, -1),
      W_yr, b_yr.reshape(1, -1))
    return out.reshape(B)
